# baseline (speedup 1.0000x reference)
import functools

import jax
import jax.numpy as jnp
from jax import lax
from jax.experimental import pallas as pl
from jax.experimental.pallas import tpu as pltpu
from jax.experimental.pallas import tpu_sc as plsc

N = 10000
E = 320000
HIDDEN = 128
NODE_VOCAB = 100

NUM_CORES = 2
NUM_SUBCORES = 16
NW = NUM_CORES * NUM_SUBCORES
EDGES_PER_W = E // NW
CHUNK = 40
NB = 10
NCHUNKS = EDGES_PER_W // CHUNK
NGROUPS = NCHUNKS // NB
EDGE_VOCAB_ROWS = 10


def _edge_gather_body(table_hbm, attr_hbm, out_hbm, idx_v, rows_v, table_sh,
                      *gsems):
    sid = lax.axis_index("s")
    wid = sid * NUM_CORES + lax.axis_index("c")
    base = wid * EDGES_PER_W
    gsem = gsems

    @pl.when(sid == 0)
    def _stage():
        pltpu.sync_copy(table_hbm, table_sh)

    pltpu.sync_copy(attr_hbm.at[pl.ds(base, EDGES_PER_W)], idx_v)
    plsc.subcore_barrier()

    def start_gather(c, b):
        pltpu.async_copy(table_sh.at[idx_v.at[pl.ds(c * CHUNK, CHUNK)]],
                         rows_v.at[b], gsem[b])

    def wait_gather(c, b):
        pltpu.make_async_copy(
            table_sh.at[idx_v.at[pl.ds(c * CHUNK, CHUNK)]],
            rows_v.at[b], gsem[b]).wait()

    for b in range(NB):
        start_gather(b, b)

    def group_body(i, carry):
        c0 = i * NB
        for b in range(NB):
            c = c0 + b
            wait_gather(c, b)
            pltpu.sync_copy(rows_v.at[b],
                            out_hbm.at[pl.ds(base + c * CHUNK, CHUNK)])

            @pl.when(c + NB < NCHUNKS)
            def _prefetch():
                start_gather(c + NB, b)

        return carry

    lax.fori_loop(0, NGROUPS, group_body, 0)

    for b in range(NCHUNKS - NGROUPS * NB):
        c = NGROUPS * NB + b
        wait_gather(c, b)
        pltpu.sync_copy(rows_v.at[b],
                        out_hbm.at[pl.ds(base + c * CHUNK, CHUNK)])


_edge_gather = functools.partial(
    pl.kernel,
    out_type=jax.ShapeDtypeStruct((E, HIDDEN), jnp.float32),
    mesh=plsc.VectorSubcoreMesh(core_axis_name="c", subcore_axis_name="s"),
    scratch_types=[
        pltpu.VMEM((EDGES_PER_W,), jnp.int32),
        pltpu.VMEM((NB, CHUNK, HIDDEN), jnp.float32),
        pltpu.VMEM_SHARED((EDGE_VOCAB_ROWS, HIDDEN), jnp.float32),
    ] + [pltpu.SemaphoreType.DMA] * NB,
)(_edge_gather_body)


ROWS_BLK = N


def _node_body(x_ref, pe_ref, nt_ref, pw_ref, pb_ref, out_ref):
    xb = x_ref[...]
    oh = (xb == lax.broadcasted_iota(jnp.int32, (ROWS_BLK, NODE_VOCAB), 1))
    node = lax.dot_general(
        oh.astype(jnp.float32), nt_ref[...],
        (((1,), (0,)), ((), ())), preferred_element_type=jnp.float32)
    proj = lax.dot_general(
        pe_ref[...], pw_ref[...],
        (((1,), (0,)), ((), ())), preferred_element_type=jnp.float32)
    out_ref[...] = node + proj + pb_ref[...]


def _node_encode(x2d, pe, node_table, pe_W, pe_b2d):
    grid = N // ROWS_BLK
    return pl.pallas_call(
        _node_body,
        grid=(grid,),
        in_specs=[
            pl.BlockSpec((ROWS_BLK, 1), lambda i: (i, 0)),
            pl.BlockSpec((ROWS_BLK, pe.shape[1]), lambda i: (i, 0)),
            pl.BlockSpec(node_table.shape, lambda i: (0, 0)),
            pl.BlockSpec(pe_W.shape, lambda i: (0, 0)),
            pl.BlockSpec((1, HIDDEN), lambda i: (0, 0)),
        ],
        out_specs=pl.BlockSpec((ROWS_BLK, HIDDEN), lambda i: (i, 0)),
        out_shape=jax.ShapeDtypeStruct((N, HIDDEN), jnp.float32),
    )(x2d, pe, node_table, pe_W, pe_b2d)


def kernel(x, edge_attr, pe, edge_index, node_table, edge_table, pe_W, pe_b):
    del edge_index
    e_emb = _edge_gather(edge_table, edge_attr.astype(jnp.int32))
    x_emb = _node_encode(
        x.astype(jnp.int32).reshape(N, 1), pe, node_table, pe_W,
        pe_b.reshape(1, HIDDEN))
    return (x_emb, e_emb)


# speedup vs baseline: 1.0610x; 1.0610x over previous
import functools

import jax
import jax.numpy as jnp
from jax import lax
from jax.experimental import pallas as pl
from jax.experimental.pallas import tpu as pltpu
from jax.experimental.pallas import tpu_sc as plsc

N = 10000
E = 320000
HIDDEN = 128
NODE_VOCAB = 100

NUM_CORES = 2
NUM_SUBCORES = 16
NW = NUM_CORES * NUM_SUBCORES
EDGES_PER_W = E // NW
CHUNK = 80
NB = 11
NCHUNKS = EDGES_PER_W // CHUNK
NGROUPS = NCHUNKS // NB
EDGE_VOCAB_ROWS = 10


def _edge_gather_body(table_hbm, attr_hbm, out_hbm, idx_v, rows_v, table_sh,
                      *gsems):
    sid = lax.axis_index("s")
    wid = sid * NUM_CORES + lax.axis_index("c")
    base = wid * EDGES_PER_W
    gsem = gsems

    @pl.when(sid == 0)
    def _stage():
        pltpu.sync_copy(table_hbm, table_sh)

    pltpu.sync_copy(attr_hbm.at[pl.ds(base, EDGES_PER_W)], idx_v)
    plsc.subcore_barrier()

    def start_gather(c, b):
        pltpu.async_copy(table_sh.at[idx_v.at[pl.ds(c * CHUNK, CHUNK)]],
                         rows_v.at[b], gsem[b])

    def wait_gather(c, b):
        pltpu.make_async_copy(
            table_sh.at[idx_v.at[pl.ds(c * CHUNK, CHUNK)]],
            rows_v.at[b], gsem[b]).wait()

    for b in range(NB):
        start_gather(b, b)

    def group_body(i, carry):
        c0 = i * NB
        for b in range(NB):
            c = c0 + b
            wait_gather(c, b)
            pltpu.sync_copy(rows_v.at[b],
                            out_hbm.at[pl.ds(base + c * CHUNK, CHUNK)])

            @pl.when(c + NB < NCHUNKS)
            def _prefetch():
                start_gather(c + NB, b)

        return carry

    lax.fori_loop(0, NGROUPS, group_body, 0)

    for b in range(NCHUNKS - NGROUPS * NB):
        c = NGROUPS * NB + b
        wait_gather(c, b)
        pltpu.sync_copy(rows_v.at[b],
                        out_hbm.at[pl.ds(base + c * CHUNK, CHUNK)])


_edge_gather = functools.partial(
    pl.kernel,
    out_type=jax.ShapeDtypeStruct((E, HIDDEN), jnp.float32),
    mesh=plsc.VectorSubcoreMesh(core_axis_name="c", subcore_axis_name="s"),
    scratch_types=[
        pltpu.VMEM((EDGES_PER_W,), jnp.int32),
        pltpu.VMEM((NB, CHUNK, HIDDEN), jnp.float32),
        pltpu.VMEM_SHARED((EDGE_VOCAB_ROWS, HIDDEN), jnp.float32),
    ] + [pltpu.SemaphoreType.DMA] * NB,
)(_edge_gather_body)


ROWS_BLK = N


def _node_body(x_ref, pe_ref, nt_ref, pw_ref, pb_ref, out_ref):
    xb = x_ref[...]
    oh = (xb == lax.broadcasted_iota(jnp.int32, (ROWS_BLK, NODE_VOCAB), 1))
    node = lax.dot_general(
        oh.astype(jnp.float32), nt_ref[...],
        (((1,), (0,)), ((), ())), preferred_element_type=jnp.float32)
    proj = lax.dot_general(
        pe_ref[...], pw_ref[...],
        (((1,), (0,)), ((), ())), preferred_element_type=jnp.float32)
    out_ref[...] = node + proj + pb_ref[...]


def _node_encode(x2d, pe, node_table, pe_W, pe_b2d):
    grid = N // ROWS_BLK
    return pl.pallas_call(
        _node_body,
        grid=(grid,),
        in_specs=[
            pl.BlockSpec((ROWS_BLK, 1), lambda i: (i, 0)),
            pl.BlockSpec((ROWS_BLK, pe.shape[1]), lambda i: (i, 0)),
            pl.BlockSpec(node_table.shape, lambda i: (0, 0)),
            pl.BlockSpec(pe_W.shape, lambda i: (0, 0)),
            pl.BlockSpec((1, HIDDEN), lambda i: (0, 0)),
        ],
        out_specs=pl.BlockSpec((ROWS_BLK, HIDDEN), lambda i: (i, 0)),
        out_shape=jax.ShapeDtypeStruct((N, HIDDEN), jnp.float32),
    )(x2d, pe, node_table, pe_W, pe_b2d)


def kernel(x, edge_attr, pe, edge_index, node_table, edge_table, pe_W, pe_b):
    del edge_index
    e_emb = _edge_gather(edge_table, edge_attr.astype(jnp.int32))
    x_emb = _node_encode(
        x.astype(jnp.int32).reshape(N, 1), pe, node_table, pe_W,
        pe_b.reshape(1, HIDDEN))
    return (x_emb, e_emb)
